# baseline (speedup 1.0000x reference)
import jax
import jax.numpy as jnp
from jax.experimental import pallas as pl
from jax.experimental.pallas import tpu as pltpu


_BLOCK = 15000


def _copy_kernel(x_ref, o_ref):
    o_ref[...] = x_ref[...]


def kernel(x):
    rows, feat = x.shape
    return pl.pallas_call(
        _copy_kernel,
        grid=(pl.cdiv(rows, _BLOCK),),
        in_specs=[pl.BlockSpec((_BLOCK, feat), lambda i: (i, 0))],
        out_specs=pl.BlockSpec((_BLOCK, feat), lambda i: (i, 0)),
        out_shape=jax.ShapeDtypeStruct(x.shape, x.dtype),
    )(x)


# speedup vs baseline: 1.0224x; 1.0224x over previous
import jax
import jax.numpy as jnp
from jax.experimental import pallas as pl
from jax.experimental.pallas import tpu as pltpu


_BLOCK = 30000


def _copy_kernel(x_ref, o_ref):
    o_ref[...] = x_ref[...]


def kernel(x):
    rows, feat = x.shape
    return pl.pallas_call(
        _copy_kernel,
        grid=(pl.cdiv(rows, _BLOCK),),
        in_specs=[pl.BlockSpec((_BLOCK, feat), lambda i: (i, 0))],
        out_specs=pl.BlockSpec((_BLOCK, feat), lambda i: (i, 0)),
        out_shape=jax.ShapeDtypeStruct(x.shape, x.dtype),
    )(x)
